# baseline (speedup 1.0000x reference)
import functools
import jax
import jax.numpy as jnp
from jax import lax
from jax.experimental import pallas as pl
from jax.experimental.pallas import tpu as pltpu
from jax.experimental.pallas import tpu_sc as plsc

_NC = 2
_NS = 16
_NW = _NC * _NS

_BATCH = 16384
_EMB = 64
_CAT = 2 * _EMB
_BPW = _BATCH // _NW
_CHK = _BPW // 2


def _sc_gather_body(ids_hbm, cat, out, c0, c1, b0, b1, sg, sw0, sw1):
    wid = lax.axis_index("s") * _NC + lax.axis_index("c")
    base = wid * _BPW
    pltpu.sync_copy(ids_hbm.at[pl.ds(base, _CHK)], c0)
    pltpu.sync_copy(ids_hbm.at[pl.ds(base + _CHK, _CHK)], c1)

    pltpu.async_copy(cat.at[c0], b0, sg).wait()
    w0 = pltpu.async_copy(b0, out.at[pl.ds(base, _CHK), :], sw0)
    pltpu.async_copy(cat.at[c1], b1, sg).wait()
    w1 = pltpu.async_copy(b1, out.at[pl.ds(base + _CHK, _CHK), :], sw1)
    w0.wait()
    w1.wait()


def _sc_gather(ids, cat):
    mesh = plsc.VectorSubcoreMesh(
        core_axis_name="c", subcore_axis_name="s",
        num_cores=_NC, num_subcores=_NS)
    out = jax.ShapeDtypeStruct((_BATCH, _CAT), jnp.float32)
    k = pl.kernel(
        _sc_gather_body,
        out_type=out,
        mesh=mesh,
        scratch_types=[
            pltpu.VMEM((_CHK,), jnp.int32),
            pltpu.VMEM((_CHK,), jnp.int32),
            pltpu.VMEM((_CHK, _CAT), jnp.float32),
            pltpu.VMEM((_CHK, _CAT), jnp.float32),
            pltpu.SemaphoreType.DMA,
            pltpu.SemaphoreType.DMA,
            pltpu.SemaphoreType.DMA,
        ],
    )
    return k(ids, cat)


_CBLK = 4000


def _concat_body(a_ref, b_ref, out_ref):
    out_ref[:, 0:_EMB] = a_ref[...]
    out_ref[:, _EMB:] = b_ref[...]


def _concat(a, b):
    n = a.shape[0]
    grid = n // _CBLK
    return pl.pallas_call(
        _concat_body,
        grid=(grid,),
        in_specs=[pl.BlockSpec((_CBLK, _EMB), lambda i: (i, 0)),
                  pl.BlockSpec((_CBLK, _EMB), lambda i: (i, 0))],
        out_specs=pl.BlockSpec((_CBLK, _CAT), lambda i: (i, 0)),
        out_shape=jax.ShapeDtypeStruct((n, _CAT), jnp.float32),
    )(a, b)


_BLK = 1024


def _mlp_body(gu_ref, gi_ref, w1_ref, b1_ref, w2_ref, b2_ref,
              wo_ref, bo_ref, out_ref):
    h1 = jnp.dot(gu_ref[:, _EMB:], w1_ref[0:_EMB, :],
                 preferred_element_type=jnp.float32)
    h1 += jnp.dot(gi_ref[:, _EMB:], w1_ref[_EMB:, :],
                  preferred_element_type=jnp.float32)
    h1 = jnp.maximum(h1 + b1_ref[...], 0.0)
    h2 = jnp.dot(h1, w2_ref[...], preferred_element_type=jnp.float32)
    h2 = jnp.maximum(h2 + b2_ref[...], 0.0)
    mf = gu_ref[:, 0:_EMB] * gi_ref[:, 0:_EMB]
    o = jnp.dot(mf, wo_ref[0:_EMB, :], preferred_element_type=jnp.float32)
    o += jnp.dot(h2, wo_ref[_EMB:, :], preferred_element_type=jnp.float32)
    out_ref[...] = o[:, 0] + bo_ref[0]


def _mlp(gu, gi, W1, b1, W2, b2, W_out, b_out):
    n_blk = _BATCH // _BLK
    row_spec = pl.BlockSpec((_BLK, _CAT), lambda i: (i, 0))
    full = lambda s: pl.BlockSpec(s, lambda i: tuple(0 for _ in s))
    return pl.pallas_call(
        _mlp_body,
        grid=(n_blk,),
        in_specs=[
            row_spec, row_spec,
            full(W1.shape), full(b1.shape), full(W2.shape), full(b2.shape),
            full(W_out.shape), full(b_out.shape),
        ],
        out_specs=pl.BlockSpec((_BLK,), lambda i: (i,)),
        out_shape=jax.ShapeDtypeStruct((_BATCH,), jnp.float32),
    )(gu, gi, W1, b1, W2, b2, W_out, b_out)


@jax.jit
def kernel(user_ids, item_ids, user_emb_mf, item_emb_mf, user_emb_mlp,
           item_emb_mlp, W1, b1, W2, b2, W_out, b_out):
    ucat = _concat(user_emb_mf, user_emb_mlp)
    gu = _sc_gather(user_ids, ucat)
    icat = _concat(item_emb_mf, item_emb_mlp)
    gi = _sc_gather(item_ids, icat)
    return _mlp(gu, gi, W1, b1, W2, b2, W_out, b_out)


# speedup vs baseline: 1.0886x; 1.0886x over previous
import functools
import jax
import jax.numpy as jnp
from jax import lax
from jax.experimental import pallas as pl
from jax.experimental.pallas import tpu as pltpu
from jax.experimental.pallas import tpu_sc as plsc

_NC = 2
_NS = 16
_NW = _NC * _NS

_BATCH = 16384
_EMB = 64
_CAT = 2 * _EMB
_BPW = _BATCH // _NW
_CHK = _BPW // 2


def _sc_gather_body(uid_hbm, iid_hbm, t0, t1, t2, t3,
                    o0, o1, o2, o3, cu0, cu1, ci0, ci1,
                    b0, b1, sg0, sg1, sw0, sw1):
    wid = lax.axis_index("s") * _NC + lax.axis_index("c")
    base = wid * _BPW
    pltpu.sync_copy(uid_hbm.at[pl.ds(base, _CHK)], cu0)
    pltpu.sync_copy(uid_hbm.at[pl.ds(base + _CHK, _CHK)], cu1)
    pltpu.sync_copy(iid_hbm.at[pl.ds(base, _CHK)], ci0)
    pltpu.sync_copy(iid_hbm.at[pl.ds(base + _CHK, _CHK)], ci1)

    phases = [
        (t0, cu0, o0, 0), (t0, cu1, o0, _CHK),
        (t1, ci0, o1, 0), (t1, ci1, o1, _CHK),
        (t2, cu0, o2, 0), (t2, cu1, o2, _CHK),
        (t3, ci0, o3, 0), (t3, ci1, o3, _CHK),
    ]
    bufs = (b0, b1)
    gsems = (sg0, sg1)
    wsems = (sw0, sw1)
    writes = [None, None]
    for k, (tab, idx, out, off) in enumerate(phases):
        p = k % 2
        if writes[p] is not None:
            writes[p].wait()
        pltpu.async_copy(tab.at[idx], bufs[p], gsems[p]).wait()
        writes[p] = pltpu.async_copy(
            bufs[p], out.at[pl.ds(base + off, _CHK), :], wsems[p])
    writes[0].wait()
    writes[1].wait()


def _sc_gather(user_ids, item_ids, t_umf, t_imf, t_umlp, t_imlp):
    mesh = plsc.VectorSubcoreMesh(
        core_axis_name="c", subcore_axis_name="s",
        num_cores=_NC, num_subcores=_NS)
    out = jax.ShapeDtypeStruct((_BATCH, _CAT), jnp.float32)
    k = pl.kernel(
        _sc_gather_body,
        out_type=(out, out, out, out),
        mesh=mesh,
        scratch_types=[
            pltpu.VMEM((_CHK,), jnp.int32),
            pltpu.VMEM((_CHK,), jnp.int32),
            pltpu.VMEM((_CHK,), jnp.int32),
            pltpu.VMEM((_CHK,), jnp.int32),
            pltpu.VMEM((_CHK, _CAT), jnp.float32),
            pltpu.VMEM((_CHK, _CAT), jnp.float32),
            pltpu.SemaphoreType.DMA,
            pltpu.SemaphoreType.DMA,
            pltpu.SemaphoreType.DMA,
            pltpu.SemaphoreType.DMA,
        ],
    )
    return k(user_ids, item_ids, t_umf, t_imf, t_umlp, t_imlp)


_CBLK = 4000


def _concat_body(a_ref, b_ref, out_ref):
    out_ref[:, 0:_EMB] = a_ref[...]
    out_ref[:, _EMB:] = b_ref[...]


def _concat(a, b):
    n = a.shape[0]
    grid = n // _CBLK
    return pl.pallas_call(
        _concat_body,
        grid=(grid,),
        in_specs=[pl.BlockSpec((_CBLK, _EMB), lambda i: (i, 0)),
                  pl.BlockSpec((_CBLK, _EMB), lambda i: (i, 0))],
        out_specs=pl.BlockSpec((_CBLK, _CAT), lambda i: (i, 0)),
        out_shape=jax.ShapeDtypeStruct((n, _CAT), jnp.float32),
    )(a, b)


_BLK = 1024


def _mlp_body(umf_ref, imf_ref, umlp_ref, imlp_ref,
              w1_ref, b1_ref, w2_ref, b2_ref, wo_ref, bo_ref, out_ref):
    h1 = jnp.dot(umlp_ref[:, 0:_EMB], w1_ref[0:_EMB, :],
                 preferred_element_type=jnp.float32)
    h1 += jnp.dot(imlp_ref[:, 0:_EMB], w1_ref[_EMB:, :],
                  preferred_element_type=jnp.float32)
    h1 = jnp.maximum(h1 + b1_ref[...], 0.0)
    h2 = jnp.dot(h1, w2_ref[...], preferred_element_type=jnp.float32)
    h2 = jnp.maximum(h2 + b2_ref[...], 0.0)
    mf = umf_ref[:, 0:_EMB] * imf_ref[:, 0:_EMB]
    o = jnp.dot(mf, wo_ref[0:_EMB, :], preferred_element_type=jnp.float32)
    o += jnp.dot(h2, wo_ref[_EMB:, :], preferred_element_type=jnp.float32)
    out_ref[...] = o[:, 0] + bo_ref[0]


def _mlp(umf, imf, umlp, imlp, W1, b1, W2, b2, W_out, b_out):
    n_blk = _BATCH // _BLK
    row_spec = pl.BlockSpec((_BLK, _CAT), lambda i: (i, 0))
    full = lambda s: pl.BlockSpec(s, lambda i: tuple(0 for _ in s))
    return pl.pallas_call(
        _mlp_body,
        grid=(n_blk,),
        in_specs=[
            row_spec, row_spec, row_spec, row_spec,
            full(W1.shape), full(b1.shape), full(W2.shape), full(b2.shape),
            full(W_out.shape), full(b_out.shape),
        ],
        out_specs=pl.BlockSpec((_BLK,), lambda i: (i,)),
        out_shape=jax.ShapeDtypeStruct((_BATCH,), jnp.float32),
    )(umf, imf, umlp, imlp, W1, b1, W2, b2, W_out, b_out)


def _probe():
    import sys
    try:
        import reference as _r
        d = _r.setup_inputs()
        for k in ("user_ids", "user_emb_mf", "W1", "W_out"):
            print("PROBE fmt", k, d[k].format, file=sys.stderr)
    except Exception as e:
        print("PROBE ERR:", repr(e), file=sys.stderr)


try:
    if jax.devices()[0].platform != "cpu":
        _probe()
except Exception:
    pass


@jax.jit
def kernel(user_ids, item_ids, user_emb_mf, item_emb_mf, user_emb_mlp,
           item_emb_mlp, W1, b1, W2, b2, W_out, b_out):
    pad = lambda t: jnp.pad(t, ((0, 0), (0, _EMB)))
    umf, imf, umlp, imlp = _sc_gather(
        user_ids, item_ids, pad(user_emb_mf), pad(item_emb_mf),
        pad(user_emb_mlp), pad(item_emb_mlp))
    return _mlp(umf, imf, umlp, imlp, W1, b1, W2, b2, W_out, b_out)


# speedup vs baseline: 1.0896x; 1.0009x over previous
import functools
import jax
import jax.numpy as jnp
from jax import lax
from jax.experimental import pallas as pl
from jax.experimental.pallas import tpu as pltpu
from jax.experimental.pallas import tpu_sc as plsc

_NC = 2
_NS = 16
_NW = _NC * _NS

_BATCH = 16384
_EMB = 64
_CAT = 2 * _EMB
_BPW = _BATCH // _NW
_CHK = _BPW // 2


def _sc_gather_body(ids_hbm, tab, out, c0, c1, b0, b1, sg, sw0, sw1):
    wid = lax.axis_index("s") * _NC + lax.axis_index("c")
    base = wid * _BPW
    pltpu.sync_copy(ids_hbm.at[pl.ds(base, _CHK)], c0)
    pltpu.sync_copy(ids_hbm.at[pl.ds(base + _CHK, _CHK)], c1)

    pltpu.async_copy(tab.at[c0], b0, sg).wait()
    w0 = pltpu.async_copy(b0, out.at[pl.ds(base, _CHK), :], sw0)
    pltpu.async_copy(tab.at[c1], b1, sg).wait()
    w1 = pltpu.async_copy(b1, out.at[pl.ds(base + _CHK, _CHK), :], sw1)
    w0.wait()
    w1.wait()


def _sc_gather(ids, tab):
    mesh = plsc.VectorSubcoreMesh(
        core_axis_name="c", subcore_axis_name="s",
        num_cores=_NC, num_subcores=_NS)
    out = jax.ShapeDtypeStruct((_BATCH, _CAT), jnp.float32)
    k = pl.kernel(
        _sc_gather_body,
        out_type=out,
        mesh=mesh,
        scratch_types=[
            pltpu.VMEM((_CHK,), jnp.int32),
            pltpu.VMEM((_CHK,), jnp.int32),
            pltpu.VMEM((_CHK, _CAT), jnp.float32),
            pltpu.VMEM((_CHK, _CAT), jnp.float32),
            pltpu.SemaphoreType.DMA,
            pltpu.SemaphoreType.DMA,
            pltpu.SemaphoreType.DMA,
        ],
    )
    return k(ids, tab)


_CBLK = 4000


def _concat_body(a_ref, b_ref, out_ref):
    out_ref[:, 0:_EMB] = a_ref[...]
    out_ref[:, _EMB:] = b_ref[...]


def _concat(a, b):
    n = a.shape[0]
    grid = n // _CBLK
    return pl.pallas_call(
        _concat_body,
        grid=(grid,),
        in_specs=[pl.BlockSpec((_CBLK, _EMB), lambda i: (i, 0)),
                  pl.BlockSpec((_CBLK, _EMB), lambda i: (i, 0))],
        out_specs=pl.BlockSpec((_CBLK, _CAT), lambda i: (i, 0)),
        out_shape=jax.ShapeDtypeStruct((n, _CAT), jnp.float32),
    )(a, b)


_BLK = 1024


def _mlp_body(umf_ref, imf_ref, umlp_ref, imlp_ref,
              w1_ref, b1_ref, w2_ref, b2_ref, wo_ref, bo_ref, out_ref):
    h1 = jnp.dot(umlp_ref[:, 0:_EMB], w1_ref[0:_EMB, :],
                 preferred_element_type=jnp.float32)
    h1 += jnp.dot(imlp_ref[:, 0:_EMB], w1_ref[_EMB:, :],
                  preferred_element_type=jnp.float32)
    h1 = jnp.maximum(h1 + b1_ref[...], 0.0)
    h2 = jnp.dot(h1, w2_ref[...], preferred_element_type=jnp.float32)
    h2 = jnp.maximum(h2 + b2_ref[...], 0.0)
    mf = umf_ref[:, 0:_EMB] * imf_ref[:, 0:_EMB]
    o = jnp.dot(mf, wo_ref[0:_EMB, :], preferred_element_type=jnp.float32)
    o += jnp.dot(h2, wo_ref[_EMB:, :], preferred_element_type=jnp.float32)
    out_ref[...] = o[:, 0] + bo_ref[0]


def _mlp(umf, imf, umlp, imlp, W1, b1, W2, b2, W_out, b_out):
    n_blk = _BATCH // _BLK
    row_spec = pl.BlockSpec((_BLK, _CAT), lambda i: (i, 0))
    full = lambda s: pl.BlockSpec(s, lambda i: tuple(0 for _ in s))
    return pl.pallas_call(
        _mlp_body,
        grid=(n_blk,),
        in_specs=[
            row_spec, row_spec, row_spec, row_spec,
            full(W1.shape), full(b1.shape), full(W2.shape), full(b2.shape),
            full(W_out.shape), full(b_out.shape),
        ],
        out_specs=pl.BlockSpec((_BLK,), lambda i: (i,)),
        out_shape=jax.ShapeDtypeStruct((_BATCH,), jnp.float32),
    )(umf, imf, umlp, imlp, W1, b1, W2, b2, W_out, b_out)


def _probe():
    import sys
    try:
        import reference as _r
        d = _r.setup_inputs()
        args = (d["user_ids"], d["item_ids"], d["user_emb_mf"],
                d["item_emb_mf"], d["user_emb_mlp"], d["item_emb_mlp"],
                d["W1"], d["b1"], d["W2"], d["b2"], d["W_out"], d["b_out"])
        txt = jax.jit(kernel).lower(*args).compile().as_text()
        with open("hlo_real.txt", "w") as f:
            f.write(txt)
        print("PROBE dumped hlo_real", file=sys.stderr)
    except Exception as e:
        print("PROBE ERR:", repr(e), file=sys.stderr)


@jax.jit
def kernel(user_ids, item_ids, user_emb_mf, item_emb_mf, user_emb_mlp,
           item_emb_mlp, W1, b1, W2, b2, W_out, b_out):
    pad = lambda t: jnp.pad(t, ((0, 0), (0, _EMB)))
    umf = _sc_gather(user_ids, pad(user_emb_mf))
    imf = _sc_gather(item_ids, pad(item_emb_mf))
    umlp = _sc_gather(user_ids, pad(user_emb_mlp))
    imlp = _sc_gather(item_ids, pad(item_emb_mlp))
    return _mlp(umf, imf, umlp, imlp, W1, b1, W2, b2, W_out, b_out)


try:
    if jax.devices()[0].platform != "cpu":
        _probe()
except Exception:
    pass


# speedup vs baseline: 1.2906x; 1.1845x over previous
import functools
import jax
import jax.numpy as jnp
from jax import lax
from jax.experimental import pallas as pl
from jax.experimental.pallas import tpu as pltpu
from jax.experimental.pallas import tpu_sc as plsc

_NC = 2
_NS = 16
_NW = _NC * _NS

_BATCH = 16384
_EMB = 64
_CAT = 2 * _EMB
_BPW = _BATCH // _NW
_CHK = _BPW // 2


def _sc_gather_body(ids_hbm, tab, out, c0, c1, b0, b1, sg, sw0, sw1):
    wid = lax.axis_index("s") * _NC + lax.axis_index("c")
    base = wid * _BPW
    pltpu.sync_copy(ids_hbm.at[pl.ds(base, _CHK)], c0)
    pltpu.sync_copy(ids_hbm.at[pl.ds(base + _CHK, _CHK)], c1)

    pltpu.async_copy(tab.at[c0], b0, sg).wait()
    w0 = pltpu.async_copy(b0, out.at[pl.ds(base, _CHK), :], sw0)
    pltpu.async_copy(tab.at[c1], b1, sg).wait()
    w1 = pltpu.async_copy(b1, out.at[pl.ds(base + _CHK, _CHK), :], sw1)
    w0.wait()
    w1.wait()


def _sc_gather(ids, tab):
    mesh = plsc.VectorSubcoreMesh(
        core_axis_name="c", subcore_axis_name="s",
        num_cores=_NC, num_subcores=_NS)
    out = jax.ShapeDtypeStruct((_BATCH, _CAT), jnp.float32)
    k = pl.kernel(
        _sc_gather_body,
        out_type=out,
        mesh=mesh,
        scratch_types=[
            pltpu.VMEM((_CHK,), jnp.int32),
            pltpu.VMEM((_CHK,), jnp.int32),
            pltpu.VMEM((_CHK, _CAT), jnp.float32),
            pltpu.VMEM((_CHK, _CAT), jnp.float32),
            pltpu.SemaphoreType.DMA,
            pltpu.SemaphoreType.DMA,
            pltpu.SemaphoreType.DMA,
        ],
    )
    return k(ids, tab)


_CBLK = 1024


def _pack_body(at_ref, bt_ref, out_ref):
    out_ref[:, 0:_EMB] = at_ref[...].T
    out_ref[:, _EMB:] = bt_ref[...].T


def _pack(at, bt):
    n = at.shape[1]
    grid = pl.cdiv(n, _CBLK)
    return pl.pallas_call(
        _pack_body,
        grid=(grid,),
        in_specs=[pl.BlockSpec((_EMB, _CBLK), lambda i: (0, i)),
                  pl.BlockSpec((_EMB, _CBLK), lambda i: (0, i))],
        out_specs=pl.BlockSpec((_CBLK, _CAT), lambda i: (i, 0)),
        out_shape=jax.ShapeDtypeStruct((n, _CAT), jnp.float32),
    )(at, bt)


_BLK = 1024


def _mlp_body(gu_ref, gi_ref, w1_ref, b1_ref, w2_ref, b2_ref,
              wo_ref, bo_ref, out_ref):
    h1 = jnp.dot(gu_ref[:, _EMB:], w1_ref[0:_EMB, :],
                 preferred_element_type=jnp.float32)
    h1 += jnp.dot(gi_ref[:, _EMB:], w1_ref[_EMB:, :],
                  preferred_element_type=jnp.float32)
    h1 = jnp.maximum(h1 + b1_ref[...], 0.0)
    h2 = jnp.dot(h1, w2_ref[...], preferred_element_type=jnp.float32)
    h2 = jnp.maximum(h2 + b2_ref[...], 0.0)
    mf = gu_ref[:, 0:_EMB] * gi_ref[:, 0:_EMB]
    o = jnp.dot(mf, wo_ref[0:_EMB, :], preferred_element_type=jnp.float32)
    o += jnp.dot(h2, wo_ref[_EMB:, :], preferred_element_type=jnp.float32)
    out_ref[...] = o[:, 0] + bo_ref[0]


def _mlp(gu, gi, W1, b1, W2, b2, W_out, b_out):
    n_blk = _BATCH // _BLK
    row_spec = pl.BlockSpec((_BLK, _CAT), lambda i: (i, 0))
    full = lambda s: pl.BlockSpec(s, lambda i: tuple(0 for _ in s))
    return pl.pallas_call(
        _mlp_body,
        grid=(n_blk,),
        in_specs=[
            row_spec, row_spec,
            full(W1.shape), full(b1.shape), full(W2.shape), full(b2.shape),
            full(W_out.shape), full(b_out.shape),
        ],
        out_specs=pl.BlockSpec((_BLK,), lambda i: (i,)),
        out_shape=jax.ShapeDtypeStruct((_BATCH,), jnp.float32),
    )(gu, gi, W1, b1, W2, b2, W_out, b_out)


def _probe():
    import sys
    try:
        import reference as _r
        d = _r.setup_inputs()
        args = (d["user_ids"], d["item_ids"], d["user_emb_mf"],
                d["item_emb_mf"], d["user_emb_mlp"], d["item_emb_mlp"],
                d["W1"], d["b1"], d["W2"], d["b2"], d["W_out"], d["b_out"])
        txt = jax.jit(kernel).lower(*args).compile().as_text()
        with open("hlo_real.txt", "w") as f:
            f.write(txt)
        print("PROBE dumped hlo_real", file=sys.stderr)
    except Exception as e:
        print("PROBE ERR:", repr(e), file=sys.stderr)


@jax.jit
def kernel(user_ids, item_ids, user_emb_mf, item_emb_mf, user_emb_mlp,
           item_emb_mlp, W1, b1, W2, b2, W_out, b_out):
    ucat = _pack(user_emb_mf.T, user_emb_mlp.T)
    gu = _sc_gather(user_ids, ucat)
    icat = _pack(item_emb_mf.T, item_emb_mlp.T)
    gi = _sc_gather(item_ids, icat)
    return _mlp(gu, gi, W1, b1, W2, b2, W_out, b_out)


try:
    if jax.devices()[0].platform != "cpu":
        _probe()
except Exception:
    pass


# speedup vs baseline: 1.6313x; 1.2639x over previous
import functools
import jax
import jax.numpy as jnp
from jax import lax
from jax.experimental import pallas as pl
from jax.experimental.pallas import tpu as pltpu
from jax.experimental.pallas import tpu_sc as plsc

_NC = 2
_NS = 16
_NW = _NC * _NS

_BATCH = 16384
_EMB = 64
_CAT = 2 * _EMB
_BPW = _BATCH // _NW
_CHK = _BPW // 2


def _sc_gather_body(ids_hbm, tab, out, c0, c1, b0, b1, sg, sw0, sw1):
    wid = lax.axis_index("s") * _NC + lax.axis_index("c")
    base = wid * _BPW
    pltpu.sync_copy(ids_hbm.at[pl.ds(base, _CHK)], c0)
    pltpu.sync_copy(ids_hbm.at[pl.ds(base + _CHK, _CHK)], c1)

    pltpu.async_copy(tab.at[c0], b0, sg).wait()
    w0 = pltpu.async_copy(b0, out.at[pl.ds(base, _CHK), :], sw0)
    pltpu.async_copy(tab.at[c1], b1, sg).wait()
    w1 = pltpu.async_copy(b1, out.at[pl.ds(base + _CHK, _CHK), :], sw1)
    w0.wait()
    w1.wait()


def _sc_gather(ids, tab):
    mesh = plsc.VectorSubcoreMesh(
        core_axis_name="c", subcore_axis_name="s",
        num_cores=_NC, num_subcores=_NS)
    out = jax.ShapeDtypeStruct((_BATCH, _CAT), jnp.float32)
    k = pl.kernel(
        _sc_gather_body,
        out_type=out,
        mesh=mesh,
        scratch_types=[
            pltpu.VMEM((_CHK,), jnp.int32),
            pltpu.VMEM((_CHK,), jnp.int32),
            pltpu.VMEM((_CHK, _CAT), jnp.float32),
            pltpu.VMEM((_CHK, _CAT), jnp.float32),
            pltpu.SemaphoreType.DMA,
            pltpu.SemaphoreType.DMA,
            pltpu.SemaphoreType.DMA,
        ],
    )
    return k(ids, tab)


_CBLK = 2048


def _pack_body(at_ref, bt_ref, out_ref):
    eye = jnp.eye(_EMB, dtype=jnp.float32)
    dn = (((0,), (0,)), ((), ()))
    out_ref[:, 0:_EMB] = lax.dot_general(
        at_ref[...], eye, dn, preferred_element_type=jnp.float32)
    out_ref[:, _EMB:] = lax.dot_general(
        bt_ref[...], eye, dn, preferred_element_type=jnp.float32)


def _pack(at, bt):
    n = at.shape[1]
    grid = pl.cdiv(n, _CBLK)
    return pl.pallas_call(
        _pack_body,
        grid=(grid,),
        in_specs=[pl.BlockSpec((_EMB, _CBLK), lambda i: (0, i)),
                  pl.BlockSpec((_EMB, _CBLK), lambda i: (0, i))],
        out_specs=pl.BlockSpec((_CBLK, _CAT), lambda i: (i, 0)),
        out_shape=jax.ShapeDtypeStruct((n, _CAT), jnp.float32),
    )(at, bt)


_BLK = 1024


def _mlp_body(gu_ref, gi_ref, w1_ref, b1_ref, w2_ref, b2_ref,
              wo_ref, bo_ref, out_ref):
    h1 = jnp.dot(gu_ref[:, _EMB:], w1_ref[0:_EMB, :],
                 preferred_element_type=jnp.float32)
    h1 += jnp.dot(gi_ref[:, _EMB:], w1_ref[_EMB:, :],
                  preferred_element_type=jnp.float32)
    h1 = jnp.maximum(h1 + b1_ref[...], 0.0)
    h2 = jnp.dot(h1, w2_ref[...], preferred_element_type=jnp.float32)
    h2 = jnp.maximum(h2 + b2_ref[...], 0.0)
    mf = gu_ref[:, 0:_EMB] * gi_ref[:, 0:_EMB]
    o = jnp.dot(mf, wo_ref[0:_EMB, :], preferred_element_type=jnp.float32)
    o += jnp.dot(h2, wo_ref[_EMB:, :], preferred_element_type=jnp.float32)
    out_ref[...] = o[:, 0] + bo_ref[0]


def _mlp(gu, gi, W1, b1, W2, b2, W_out, b_out):
    n_blk = _BATCH // _BLK
    row_spec = pl.BlockSpec((_BLK, _CAT), lambda i: (i, 0))
    full = lambda s: pl.BlockSpec(s, lambda i: tuple(0 for _ in s))
    return pl.pallas_call(
        _mlp_body,
        grid=(n_blk,),
        in_specs=[
            row_spec, row_spec,
            full(W1.shape), full(b1.shape), full(W2.shape), full(b2.shape),
            full(W_out.shape), full(b_out.shape),
        ],
        out_specs=pl.BlockSpec((_BLK,), lambda i: (i,)),
        out_shape=jax.ShapeDtypeStruct((_BATCH,), jnp.float32),
    )(gu, gi, W1, b1, W2, b2, W_out, b_out)


def _probe():
    import sys
    try:
        import reference as _r
        d = _r.setup_inputs()
        args = (d["user_ids"], d["item_ids"], d["user_emb_mf"],
                d["item_emb_mf"], d["user_emb_mlp"], d["item_emb_mlp"],
                d["W1"], d["b1"], d["W2"], d["b2"], d["W_out"], d["b_out"])
        txt = jax.jit(kernel).lower(*args).compile().as_text()
        with open("hlo_real.txt", "w") as f:
            f.write(txt)
        print("PROBE dumped hlo_real", file=sys.stderr)
    except Exception as e:
        print("PROBE ERR:", repr(e), file=sys.stderr)


@jax.jit
def kernel(user_ids, item_ids, user_emb_mf, item_emb_mf, user_emb_mlp,
           item_emb_mlp, W1, b1, W2, b2, W_out, b_out):
    ucat = _pack(user_emb_mf.T, user_emb_mlp.T)
    gu = _sc_gather(user_ids, ucat)
    icat = _pack(item_emb_mf.T, item_emb_mlp.T)
    gi = _sc_gather(item_ids, icat)
    return _mlp(gu, gi, W1, b1, W2, b2, W_out, b_out)


try:
    if jax.devices()[0].platform != "cpu":
        _probe()
except Exception:
    pass


# speedup vs baseline: 1.9349x; 1.1861x over previous
import functools
import jax
import jax.numpy as jnp
from jax import lax
from jax.experimental import pallas as pl
from jax.experimental.pallas import tpu as pltpu
from jax.experimental.pallas import tpu_sc as plsc

_NC = 2
_NS = 16
_NW = _NC * _NS

_BATCH = 16384
_EMB = 64
_CAT = 2 * _EMB
_BPW = _BATCH // _NW
_CHK = _BPW // 2


def _sc_gather_body(ids_hbm, tab, out, c0, c1, b0, b1, sg, sw0, sw1):
    wid = lax.axis_index("s") * _NC + lax.axis_index("c")
    base = wid * _BPW
    pltpu.sync_copy(ids_hbm.at[pl.ds(base, _CHK)], c0)
    pltpu.sync_copy(ids_hbm.at[pl.ds(base + _CHK, _CHK)], c1)

    pltpu.async_copy(tab.at[c0], b0, sg).wait()
    w0 = pltpu.async_copy(b0, out.at[pl.ds(base, _CHK), :], sw0)
    pltpu.async_copy(tab.at[c1], b1, sg).wait()
    w1 = pltpu.async_copy(b1, out.at[pl.ds(base + _CHK, _CHK), :], sw1)
    w0.wait()
    w1.wait()


def _sc_gather(ids, tab):
    mesh = plsc.VectorSubcoreMesh(
        core_axis_name="c", subcore_axis_name="s",
        num_cores=_NC, num_subcores=_NS)
    out = jax.ShapeDtypeStruct((_BATCH, _CAT), jnp.float32)
    k = pl.kernel(
        _sc_gather_body,
        out_type=out,
        mesh=mesh,
        scratch_types=[
            pltpu.VMEM((_CHK,), jnp.int32),
            pltpu.VMEM((_CHK,), jnp.int32),
            pltpu.VMEM((_CHK, _CAT), jnp.float32),
            pltpu.VMEM((_CHK, _CAT), jnp.float32),
            pltpu.SemaphoreType.DMA,
            pltpu.SemaphoreType.DMA,
            pltpu.SemaphoreType.DMA,
        ],
    )
    return k(ids, tab)


_CBLK = 4096


def _pack_body(at_ref, bt_ref, out_ref):
    eye = jnp.eye(_EMB, dtype=jnp.float32)
    dn = (((0,), (0,)), ((), ()))
    out_ref[:, 0:_EMB] = lax.dot_general(
        at_ref[...], eye, dn, preferred_element_type=jnp.float32)
    out_ref[:, _EMB:] = lax.dot_general(
        bt_ref[...], eye, dn, preferred_element_type=jnp.float32)


def _pack(at, bt):
    n = at.shape[1]
    grid = pl.cdiv(n, _CBLK)
    return pl.pallas_call(
        _pack_body,
        grid=(grid,),
        in_specs=[pl.BlockSpec((_EMB, _CBLK), lambda i: (0, i)),
                  pl.BlockSpec((_EMB, _CBLK), lambda i: (0, i))],
        out_specs=pl.BlockSpec((_CBLK, _CAT), lambda i: (i, 0)),
        out_shape=jax.ShapeDtypeStruct((n, _CAT), jnp.float32),
    )(at, bt)


_BLK = 2048


def _mlp_body(gu_ref, gi_ref, w1_ref, b1_ref, w2_ref, b2_ref,
              wo_ref, bo_ref, out_ref):
    h1 = jnp.dot(gu_ref[:, _EMB:], w1_ref[0:_EMB, :],
                 preferred_element_type=jnp.float32)
    h1 += jnp.dot(gi_ref[:, _EMB:], w1_ref[_EMB:, :],
                  preferred_element_type=jnp.float32)
    h1 = jnp.maximum(h1 + b1_ref[...], 0.0)
    h2 = jnp.dot(h1, w2_ref[...], preferred_element_type=jnp.float32)
    h2 = jnp.maximum(h2 + b2_ref[...], 0.0)
    mf = gu_ref[:, 0:_EMB] * gi_ref[:, 0:_EMB]
    o = jnp.dot(mf, wo_ref[0:_EMB, :], preferred_element_type=jnp.float32)
    o += jnp.dot(h2, wo_ref[_EMB:, :], preferred_element_type=jnp.float32)
    out_ref[...] = o[:, 0] + bo_ref[0]


def _mlp(gu, gi, W1, b1, W2, b2, W_out, b_out):
    n_blk = _BATCH // _BLK
    row_spec = pl.BlockSpec((_BLK, _CAT), lambda i: (i, 0))
    full = lambda s: pl.BlockSpec(s, lambda i: tuple(0 for _ in s))
    return pl.pallas_call(
        _mlp_body,
        grid=(n_blk,),
        in_specs=[
            row_spec, row_spec,
            full(W1.shape), full(b1.shape), full(W2.shape), full(b2.shape),
            full(W_out.shape), full(b_out.shape),
        ],
        out_specs=pl.BlockSpec((_BLK,), lambda i: (i,)),
        out_shape=jax.ShapeDtypeStruct((_BATCH,), jnp.float32),
    )(gu, gi, W1, b1, W2, b2, W_out, b_out)


@jax.jit
def kernel(user_ids, item_ids, user_emb_mf, item_emb_mf, user_emb_mlp,
           item_emb_mlp, W1, b1, W2, b2, W_out, b_out):
    ucat = _pack(user_emb_mf.T, user_emb_mlp.T)
    gu = _sc_gather(user_ids, ucat)
    icat = _pack(item_emb_mf.T, item_emb_mlp.T)
    gi = _sc_gather(item_ids, icat)
    return _mlp(gu, gi, W1, b1, W2, b2, W_out, b_out)


# speedup vs baseline: 2.0671x; 1.0683x over previous
import functools
import jax
import jax.numpy as jnp
from jax import lax
from jax.experimental import pallas as pl
from jax.experimental.pallas import tpu as pltpu
from jax.experimental.pallas import tpu_sc as plsc

_NC = 2
_NS = 16
_NW = _NC * _NS

_BATCH = 16384
_EMB = 64
_CAT = 2 * _EMB
_BPW = _BATCH // _NW
_CHK = _BPW // 2


def _sc_gather_body(ids_hbm, tab, out, c0, c1, b0, b1, sg, sw0, sw1):
    wid = lax.axis_index("s") * _NC + lax.axis_index("c")
    base = wid * _BPW
    pltpu.sync_copy(ids_hbm.at[pl.ds(base, _CHK)], c0)
    pltpu.sync_copy(ids_hbm.at[pl.ds(base + _CHK, _CHK)], c1)

    pltpu.async_copy(tab.at[c0], b0, sg).wait()
    w0 = pltpu.async_copy(b0, out.at[pl.ds(base, _CHK), :], sw0)
    pltpu.async_copy(tab.at[c1], b1, sg).wait()
    w1 = pltpu.async_copy(b1, out.at[pl.ds(base + _CHK, _CHK), :], sw1)
    w0.wait()
    w1.wait()


def _sc_gather(ids, tab):
    mesh = plsc.VectorSubcoreMesh(
        core_axis_name="c", subcore_axis_name="s",
        num_cores=_NC, num_subcores=_NS)
    out = jax.ShapeDtypeStruct((_BATCH, _CAT), jnp.float32)
    k = pl.kernel(
        _sc_gather_body,
        out_type=out,
        mesh=mesh,
        scratch_types=[
            pltpu.VMEM((_CHK,), jnp.int32),
            pltpu.VMEM((_CHK,), jnp.int32),
            pltpu.VMEM((_CHK, _CAT), jnp.float32),
            pltpu.VMEM((_CHK, _CAT), jnp.float32),
            pltpu.SemaphoreType.DMA,
            pltpu.SemaphoreType.DMA,
            pltpu.SemaphoreType.DMA,
        ],
    )
    return k(ids, tab)


_CBLK = 8192


def _pack_body(at_ref, bt_ref, out_ref):
    eye = jnp.eye(_EMB, dtype=jnp.float32)
    dn = (((0,), (0,)), ((), ()))
    out_ref[:, 0:_EMB] = lax.dot_general(
        at_ref[...], eye, dn, preferred_element_type=jnp.float32)
    out_ref[:, _EMB:] = lax.dot_general(
        bt_ref[...], eye, dn, preferred_element_type=jnp.float32)


def _pack(at, bt):
    n = at.shape[1]
    grid = pl.cdiv(n, _CBLK)
    return pl.pallas_call(
        _pack_body,
        grid=(grid,),
        in_specs=[pl.BlockSpec((_EMB, _CBLK), lambda i: (0, i)),
                  pl.BlockSpec((_EMB, _CBLK), lambda i: (0, i))],
        out_specs=pl.BlockSpec((_CBLK, _CAT), lambda i: (i, 0)),
        out_shape=jax.ShapeDtypeStruct((n, _CAT), jnp.float32),
    )(at, bt)


_BLK = 4096


def _mlp_body(gu_ref, gi_ref, w1_ref, b1_ref, w2_ref, b2_ref,
              wo_ref, bo_ref, out_ref):
    h1 = jnp.dot(gu_ref[:, _EMB:], w1_ref[0:_EMB, :],
                 preferred_element_type=jnp.float32)
    h1 += jnp.dot(gi_ref[:, _EMB:], w1_ref[_EMB:, :],
                  preferred_element_type=jnp.float32)
    h1 = jnp.maximum(h1 + b1_ref[...], 0.0)
    h2 = jnp.dot(h1, w2_ref[...], preferred_element_type=jnp.float32)
    h2 = jnp.maximum(h2 + b2_ref[...], 0.0)
    mf = gu_ref[:, 0:_EMB] * gi_ref[:, 0:_EMB]
    o = jnp.dot(mf, wo_ref[0:_EMB, :], preferred_element_type=jnp.float32)
    o += jnp.dot(h2, wo_ref[_EMB:, :], preferred_element_type=jnp.float32)
    out_ref[...] = o[:, 0] + bo_ref[0]


def _mlp(gu, gi, W1, b1, W2, b2, W_out, b_out):
    n_blk = _BATCH // _BLK
    row_spec = pl.BlockSpec((_BLK, _CAT), lambda i: (i, 0))
    full = lambda s: pl.BlockSpec(s, lambda i: tuple(0 for _ in s))
    return pl.pallas_call(
        _mlp_body,
        grid=(n_blk,),
        in_specs=[
            row_spec, row_spec,
            full(W1.shape), full(b1.shape), full(W2.shape), full(b2.shape),
            full(W_out.shape), full(b_out.shape),
        ],
        out_specs=pl.BlockSpec((_BLK,), lambda i: (i,)),
        out_shape=jax.ShapeDtypeStruct((_BATCH,), jnp.float32),
    )(gu, gi, W1, b1, W2, b2, W_out, b_out)


@jax.jit
def kernel(user_ids, item_ids, user_emb_mf, item_emb_mf, user_emb_mlp,
           item_emb_mlp, W1, b1, W2, b2, W_out, b_out):
    ucat = _pack(user_emb_mf.T, user_emb_mlp.T)
    gu = _sc_gather(user_ids, ucat)
    icat = _pack(item_emb_mf.T, item_emb_mlp.T)
    gi = _sc_gather(item_ids, icat)
    return _mlp(gu, gi, W1, b1, W2, b2, W_out, b_out)
